# baseline (speedup 1.0000x reference)
import jax
import jax.numpy as jnp
from jax import lax
from jax.experimental import pallas as pl
from jax.experimental.pallas import tpu as pltpu
from jax.experimental.pallas import tpu_sc as plsc

P, B, A, T, S, H = 32, 256, 32, 32, 32, 16384
N = B * A
L = 16
PI = 3.14159265358979

_ACOS_C = (1.5707963050, -0.2145988016, 0.0889789874, -0.0501743046,
           0.0308918810, -0.0170881256, 0.0066700901, -0.0012624911)


def _iota():
    return lax.iota(jnp.int32, L)


def _splat_i(x):
    return jnp.broadcast_to(jnp.asarray(x, jnp.int32), (L,))


def _vrsqrt(x):
    i = lax.bitcast_convert_type(x, jnp.int32)
    y = lax.bitcast_convert_type(jnp.int32(0x5F3759DF) - (i >> 1), jnp.float32)
    half = jnp.float32(0.5) * x
    for _ in range(2):
        y = y * (jnp.float32(1.5) - half * y * y)
    return y


def _vsqrt(x):
    return x * _vrsqrt(x)


def _vacos(c):
    t = jnp.abs(c)
    p = jnp.full((L,), _ACOS_C[7], jnp.float32)
    for a in _ACOS_C[6::-1]:
        p = p * t + jnp.float32(a)
    pos = _vsqrt(jnp.float32(1.0) - t) * p
    return jnp.where(c >= 0, pos, jnp.float32(PI) - pos)


def _cossin_2piv(v):
    a = v * jnp.float32(4.0)
    q = a.astype(jnp.int32)
    z = (a - q.astype(jnp.float32)) * jnp.float32(PI / 2)
    z2 = z * z
    c0 = jnp.float32(1.0) + z2 * (jnp.float32(-0.5) + z2 * (
        jnp.float32(1.0 / 24) + z2 * (jnp.float32(-1.0 / 720)
                                      + z2 * jnp.float32(1.0 / 40320))))
    s0 = z * (jnp.float32(1.0) + z2 * (jnp.float32(-1.0 / 6) + z2 * (
        jnp.float32(1.0 / 120) + z2 * (jnp.float32(-1.0 / 5040)
                                       + z2 * jnp.float32(1.0 / 362880)))))
    q1, q2, q3 = q == 1, q == 2, q == 3
    cos = jnp.where(q1, -s0, jnp.where(q2, -c0, jnp.where(q3, s0, c0)))
    sin = jnp.where(q1, c0, jnp.where(q2, -s0, jnp.where(q3, -c0, s0)))
    return cos, sin


def _body(coords_h, bt_h, conns_h, offs_h, subs_h, uid_h, wid_h, paths_h,
          cnt_h, hkeys_h, hvals_h, out_h,
          coords_v, bt_v, conns_v, offs_v, subs_v, uid_v, wid_v, paths_v,
          cnt_v, hk_v, ku_v, hu_v, hw_v, pu_v, pw_v, prmT_v, res_v, sem):
    wid = lax.axis_index("s") * 2 + lax.axis_index("c")

    pltpu.sync_copy(coords_h.at[wid], coords_v)
    pltpu.sync_copy(bt_h.at[wid], bt_v)
    pltpu.sync_copy(conns_h.at[wid], conns_v)
    pltpu.sync_copy(offs_h.at[wid], offs_v)
    pltpu.sync_copy(subs_h, subs_v)
    pltpu.sync_copy(uid_h, uid_v)
    pltpu.sync_copy(wid_h, wid_v)
    pltpu.sync_copy(paths_h, paths_v)
    pltpu.sync_copy(cnt_h, cnt_v)
    pltpu.sync_copy(hkeys_h, hk_v)

    lanes = _iota()
    eps = jnp.float32(1e-6)

    def keys_body(i, carry):
        ts = i * L + lanes
        t = ts >> 5
        s = ts & 31
        ku = _splat_i(0)
        kw = _splat_i(0)
        for k, mult in enumerate((131, 31, 7, 1)):
            sub_k = plsc.load_gather(subs_v, [t, s, _splat_i(k)])
            ku = ku + plsc.load_gather(uid_v, [t, sub_k]) * mult
            kw = kw + plsc.load_gather(wid_v, [t, sub_k]) * mult
        plsc.store_scatter(ku_v, [ts], ku)
        plsc.store_scatter(hu_v, [ts], ku & (H - 1))
        plsc.store_scatter(hw_v, [ts], kw & (H - 1))
        return carry

    lax.fori_loop(0, (T * S) // L, keys_body, 0)

    copies = []
    for j in range(8):
        sl = pl.ds(j * 128, 128)
        copies.append(pltpu.async_copy(hvals_h.at[hu_v.at[sl]], pu_v.at[sl], sem))
        copies.append(pltpu.async_copy(hvals_h.at[hw_v.at[sl]], pw_v.at[sl], sem))
    for c in copies:
        c.wait()

    def prm_body(i, carry):
        ts = i * L + lanes
        t = ts >> 5
        s = ts & 31
        ku = plsc.load_gather(ku_v, [ts])
        hu = plsc.load_gather(hu_v, [ts])
        match = plsc.load_gather(hk_v, [hu]) == ku
        prm = []
        for c in range(6):
            pu_c = plsc.load_gather(pu_v, [ts, _splat_i(c)])
            pw_c = plsc.load_gather(pw_v, [ts, _splat_i(c)])
            prm.append(jnp.where(match, pu_c, pw_c))
        cp0, sp0 = _cossin_2piv(prm[5])
        cntv = plsc.load_gather(cnt_v, [t])
        maskf = jnp.where(s < cntv, jnp.float32(1.0), jnp.float32(0.0))
        rows = (prm[0], prm[1] * jnp.float32(2.0), prm[2],
                prm[3] * jnp.float32(PI), prm[4], cp0, sp0, maskf)
        for c, val in enumerate(rows):
            plsc.store_scatter(prmT_v, [c * 1024 + ts], val)
        return carry

    lax.fori_loop(0, (T * S) // L, prm_body, 0)

    def intra_body(q, acc):
        b = q >> 1
        s0 = (q & 1) * L
        bvec = jnp.broadcast_to(b, (L,))
        tvec = plsc.load_gather(bt_v, [bvec])
        ovec = plsc.load_gather(offs_v, [bvec])
        svec = s0 + lanes
        ts = tvec * 32 + svec
        xs = []
        for k in range(4):
            sub_k = plsc.load_gather(subs_v, [tvec, svec, _splat_i(k)])
            gk = ovec + sub_k
            xs.append(tuple(
                plsc.load_gather(coords_v, [gk, _splat_i(c)])
                for c in range(3)))
        x0, x1, x2, x3 = xs
        prm = tuple(plsc.load_gather(prmT_v, [c * 1024 + ts])
                    for c in range(8))
        k_len, l0, k_ang, t0, k_tor, cp0, sp0, maskf = prm

        dx = tuple(x1[c] - x0[c] for c in range(3))
        d01 = _vsqrt(dx[0] * dx[0] + dx[1] * dx[1] + dx[2] * dx[2] + eps)
        uv = tuple(x0[c] - x1[c] for c in range(3))
        vv = tuple(x2[c] - x1[c] for c in range(3))
        s_uv = uv[0] * uv[0] + uv[1] * uv[1] + uv[2] * uv[2] + eps
        s_vv = vv[0] * vv[0] + vv[1] * vv[1] + vv[2] * vv[2] + eps
        dotuv = uv[0] * vv[0] + uv[1] * vv[1] + uv[2] * vv[2]
        cosang = jnp.clip(dotuv * _vrsqrt(s_uv * s_vv),
                          jnp.float32(-1.0 + 1e-6), jnp.float32(1.0 - 1e-6))
        theta = _vacos(cosang)
        b1 = dx
        b2 = vv
        b3 = tuple(x3[c] - x2[c] for c in range(3))

        def cross(u, v):
            return (u[1] * v[2] - u[2] * v[1],
                    u[2] * v[0] - u[0] * v[2],
                    u[0] * v[1] - u[1] * v[0])

        n1 = cross(b1, b2)
        n2 = cross(b2, b3)
        s_b2 = b2[0] * b2[0] + b2[1] * b2[1] + b2[2] * b2[2]
        inv_b2 = jnp.float32(1.0) / (_vsqrt(s_b2) + eps)
        m1 = cross(n1, tuple(b2[c] * inv_b2 for c in range(3)))
        y = m1[0] * n2[0] + m1[1] * n2[1] + m1[2] * n2[2]
        x = n1[0] * n2[0] + n1[1] * n2[1] + n1[2] * n2[2] + eps
        den = x * x + y * y + jnp.float32(1e-30)
        cos2phi = (x * x - y * y) / den
        sin2phi = jnp.float32(2.0) * x * y / den

        dl = d01 - l0
        da = theta - t0
        E = (k_len * dl * dl + k_ang * da * da
             + k_tor * (jnp.float32(1.0) + cos2phi * cp0 + sin2phi * sp0))
        return acc + E * maskf

    acc = lax.fori_loop(0, (B * S) // L, intra_body,
                        jnp.zeros((L,), jnp.float32))

    def inter_body(it, acc):
        e = it * L + lanes
        b = e >> 1
        j = e & 1
        zero = _splat_i(0)
        t1 = plsc.load_gather(bt_v, [b])
        b2i = plsc.load_gather(conns_v, [b, j, zero])
        c2 = plsc.load_gather(conns_v, [b, j, _splat_i(1)]) & 1
        t2 = plsc.load_gather(bt_v, [b2i])
        a1 = plsc.load_gather(paths_v, [t1, j, zero])
        a2 = plsc.load_gather(paths_v, [t2, c2, zero])
        g1 = plsc.load_gather(offs_v, [b]) + a1
        g2 = plsc.load_gather(offs_v, [b2i]) + a2
        d2 = eps
        for c in range(3):
            dc = (plsc.load_gather(coords_v, [g2, _splat_i(c)])
                  - plsc.load_gather(coords_v, [g1, _splat_i(c)]))
            d2 = d2 + dc * dc
        dd = _vsqrt(d2) - jnp.float32(1.5)
        return acc + jnp.float32(0.5) * dd * dd

    acc = lax.fori_loop(0, (B * 2) // L, inter_body, acc)

    total = jnp.sum(acc)
    res_v[...] = jnp.where(lanes == 0, jnp.broadcast_to(total, (L,)),
                           jnp.float32(0.0))
    pltpu.sync_copy(res_v, out_h.at[wid])


@jax.jit
def _run(coords, bt, conns, offs, subs, uid, wid, paths, cnts, hkeys, hvals):
    mesh = plsc.VectorSubcoreMesh(core_axis_name="c", subcore_axis_name="s")
    f = pl.kernel(
        _body,
        out_type=jax.ShapeDtypeStruct((P, L), jnp.float32),
        mesh=mesh,
        compiler_params=pltpu.CompilerParams(needs_layout_passes=False,
                                             use_tc_tiling_on_sc=False),
        scratch_types=[
            pltpu.VMEM((N, 3), jnp.float32),
            pltpu.VMEM((B,), jnp.int32),
            pltpu.VMEM((B, 2, 2), jnp.int32),
            pltpu.VMEM((B,), jnp.int32),
            pltpu.VMEM((T, S, 4), jnp.int32),
            pltpu.VMEM((T, A), jnp.int32),
            pltpu.VMEM((T, A), jnp.int32),
            pltpu.VMEM((T, 2, 3), jnp.int32),
            pltpu.VMEM((T,), jnp.int32),
            pltpu.VMEM((H,), jnp.int32),
            pltpu.VMEM((T * S,), jnp.int32),
            pltpu.VMEM((T * S,), jnp.int32),
            pltpu.VMEM((T * S,), jnp.int32),
            pltpu.VMEM((T * S, 6), jnp.float32),
            pltpu.VMEM((T * S, 6), jnp.float32),
            pltpu.VMEM((8 * T * S,), jnp.float32),
            pltpu.VMEM((L,), jnp.float32),
            pltpu.SemaphoreType.DMA,
        ],
    )
    return f(coords, bt, conns, offs, subs, uid, wid, paths, cnts, hkeys,
             hvals)


def kernel(coords, pose_stack_block_coord_offset, pose_stack_block_types,
           pose_stack_inter_block_connections, atom_paths_from_conn,
           atom_unique_ids, atom_wildcard_ids, hash_keys, hash_values,
           cart_subgraphs, cart_subgraph_offsets, max_subgraphs_per_block):
    out = _run(coords, pose_stack_block_types,
               pose_stack_inter_block_connections,
               pose_stack_block_coord_offset, cart_subgraphs,
               atom_unique_ids, atom_wildcard_ids, atom_paths_from_conn,
               cart_subgraph_offsets, hash_keys, hash_values)
    return out[:, 0]


# speedup vs baseline: 3.1450x; 3.1450x over previous
import jax
import jax.numpy as jnp
from jax import lax
from jax.experimental import pallas as pl
from jax.experimental.pallas import tpu as pltpu
from jax.experimental.pallas import tpu_sc as plsc

P, B, A, T, S, H = 32, 256, 32, 32, 32, 16384
N = B * A
L = 16
PI = 3.14159265358979

_ACOS_C = (1.5707963050, -0.2145988016, 0.0889789874, -0.0501743046,
           0.0308918810, -0.0170881256, 0.0066700901, -0.0012624911)


def _iota():
    return lax.iota(jnp.int32, L)


def _splat_i(x):
    return jnp.broadcast_to(jnp.asarray(x, jnp.int32), (L,))


def _vrsqrt(x):
    i = lax.bitcast_convert_type(x, jnp.int32)
    y = lax.bitcast_convert_type(jnp.int32(0x5F3759DF) - (i >> 1), jnp.float32)
    half = jnp.float32(0.5) * x
    for _ in range(2):
        y = y * (jnp.float32(1.5) - half * y * y)
    return y


def _vsqrt(x):
    return x * _vrsqrt(x)


def _vacos(c):
    t = jnp.abs(c)
    p = jnp.full((L,), _ACOS_C[7], jnp.float32)
    for a in _ACOS_C[6::-1]:
        p = p * t + jnp.float32(a)
    pos = _vsqrt(jnp.float32(1.0) - t) * p
    return jnp.where(c >= 0, pos, jnp.float32(PI) - pos)


def _cossin_2piv(v):
    a = v * jnp.float32(4.0)
    q = a.astype(jnp.int32)
    z = (a - q.astype(jnp.float32)) * jnp.float32(PI / 2)
    z2 = z * z
    c0 = jnp.float32(1.0) + z2 * (jnp.float32(-0.5) + z2 * (
        jnp.float32(1.0 / 24) + z2 * (jnp.float32(-1.0 / 720)
                                      + z2 * jnp.float32(1.0 / 40320))))
    s0 = z * (jnp.float32(1.0) + z2 * (jnp.float32(-1.0 / 6) + z2 * (
        jnp.float32(1.0 / 120) + z2 * (jnp.float32(-1.0 / 5040)
                                       + z2 * jnp.float32(1.0 / 362880)))))
    q1, q2, q3 = q == 1, q == 2, q == 3
    cos = jnp.where(q1, -s0, jnp.where(q2, -c0, jnp.where(q3, s0, c0)))
    sin = jnp.where(q1, c0, jnp.where(q2, -s0, jnp.where(q3, -c0, s0)))
    return cos, sin


def _gather(ref, idx):
    return plsc.load_gather(ref, [idx])


def _body(coords_h, bt_h, conns_h, offs_h, subsT_h, uid_h, wid_h, paths0_h,
          cnt_h, htab_h, out_h,
          coords_v, bt_v, conns_v, offs_v, subsT_v, uid_v, wid_v, paths0_v,
          cnt_v, ku_v, hu_v, hw_v, pu_v, pw_v, prmT_v, res_v, sem):
    wid = lax.axis_index("s") * 2 + lax.axis_index("c")

    pltpu.sync_copy(coords_h.at[wid], coords_v)
    pltpu.sync_copy(bt_h.at[wid], bt_v)
    pltpu.sync_copy(conns_h.at[wid], conns_v)
    pltpu.sync_copy(offs_h.at[wid], offs_v)
    pltpu.sync_copy(subsT_h, subsT_v)
    pltpu.sync_copy(uid_h, uid_v)
    pltpu.sync_copy(wid_h, wid_v)
    pltpu.sync_copy(paths0_h, paths0_v)
    pltpu.sync_copy(cnt_h, cnt_v)

    lanes = _iota()
    eps = jnp.float32(1e-6)

    def keys_body(i, carry):
        ts = i * L + lanes
        t32 = (ts >> 5) * 32
        ku = _splat_i(0)
        kw = _splat_i(0)
        for k, mult in enumerate((131, 31, 7, 1)):
            sub_k = _gather(subsT_v, k * 1024 + ts)
            ku = ku + _gather(uid_v, t32 + sub_k) * mult
            kw = kw + _gather(wid_v, t32 + sub_k) * mult
        plsc.store_scatter(ku_v, [ts], ku)
        plsc.store_scatter(hu_v, [ts], ku & (H - 1))
        plsc.store_scatter(hw_v, [ts], kw & (H - 1))
        return carry

    lax.fori_loop(0, (T * S) // L, keys_body, 0)

    copies = []
    for j in range(8):
        sl = pl.ds(j * 128, 128)
        copies.append(pltpu.async_copy(htab_h.at[hu_v.at[sl]], pu_v.at[sl], sem))
        copies.append(pltpu.async_copy(htab_h.at[hw_v.at[sl]], pw_v.at[sl], sem))
    for c in copies:
        c.wait()

    def prm_body(i, carry):
        ts = i * L + lanes
        t = ts >> 5
        s = ts & 31
        ku = _gather(ku_v, ts)
        key_u = lax.bitcast_convert_type(
            plsc.load_gather(pu_v, [ts, _splat_i(6)]), jnp.int32)
        match = key_u == ku
        prm = []
        for c in range(6):
            pu_c = plsc.load_gather(pu_v, [ts, _splat_i(c)])
            pw_c = plsc.load_gather(pw_v, [ts, _splat_i(c)])
            prm.append(jnp.where(match, pu_c, pw_c))
        cp0, sp0 = _cossin_2piv(prm[5])
        cntv = _gather(cnt_v, t)
        maskf = jnp.where(s < cntv, jnp.float32(1.0), jnp.float32(0.0))
        rows = (prm[0], prm[1] * jnp.float32(2.0), prm[2],
                prm[3] * jnp.float32(PI), prm[4], cp0, sp0, maskf)
        for c, val in enumerate(rows):
            plsc.store_scatter(prmT_v, [c * 1024 + ts], val)
        return carry

    lax.fori_loop(0, (T * S) // L, prm_body, 0)

    def intra_body(q, acc):
        b = q >> 1
        s0 = (q & 1) * L
        bvec = jnp.broadcast_to(b, (L,))
        tvec = _gather(bt_v, bvec)
        ovec = _gather(offs_v, bvec)
        ts = tvec * 32 + s0 + lanes
        xs = []
        for k in range(4):
            sub_k = _gather(subsT_v, k * 1024 + ts)
            gk = (ovec + sub_k) * 3
            xs.append(tuple(_gather(coords_v, gk + c) for c in range(3)))
        x0, x1, x2, x3 = xs
        prm = tuple(_gather(prmT_v, c * 1024 + ts) for c in range(8))
        k_len, l0, k_ang, t0, k_tor, cp0, sp0, maskf = prm

        dx = tuple(x1[c] - x0[c] for c in range(3))
        d01 = _vsqrt(dx[0] * dx[0] + dx[1] * dx[1] + dx[2] * dx[2] + eps)
        uv = tuple(x0[c] - x1[c] for c in range(3))
        vv = tuple(x2[c] - x1[c] for c in range(3))
        s_uv = uv[0] * uv[0] + uv[1] * uv[1] + uv[2] * uv[2] + eps
        s_vv = vv[0] * vv[0] + vv[1] * vv[1] + vv[2] * vv[2] + eps
        dotuv = uv[0] * vv[0] + uv[1] * vv[1] + uv[2] * vv[2]
        cosang = jnp.clip(dotuv * _vrsqrt(s_uv * s_vv),
                          jnp.float32(-1.0 + 1e-6), jnp.float32(1.0 - 1e-6))
        theta = _vacos(cosang)
        b1 = dx
        b2 = vv
        b3 = tuple(x3[c] - x2[c] for c in range(3))

        def cross(u, v):
            return (u[1] * v[2] - u[2] * v[1],
                    u[2] * v[0] - u[0] * v[2],
                    u[0] * v[1] - u[1] * v[0])

        n1 = cross(b1, b2)
        n2 = cross(b2, b3)
        s_b2 = b2[0] * b2[0] + b2[1] * b2[1] + b2[2] * b2[2]
        inv_b2 = jnp.float32(1.0) / (_vsqrt(s_b2) + eps)
        m1 = cross(n1, tuple(b2[c] * inv_b2 for c in range(3)))
        y = m1[0] * n2[0] + m1[1] * n2[1] + m1[2] * n2[2]
        x = n1[0] * n2[0] + n1[1] * n2[1] + n1[2] * n2[2] + eps
        den = x * x + y * y + jnp.float32(1e-30)
        cos2phi = (x * x - y * y) / den
        sin2phi = jnp.float32(2.0) * x * y / den

        dl = d01 - l0
        da = theta - t0
        E = (k_len * dl * dl + k_ang * da * da
             + k_tor * (jnp.float32(1.0) + cos2phi * cp0 + sin2phi * sp0))
        return acc + E * maskf

    acc = lax.fori_loop(0, (B * S) // L, intra_body,
                        jnp.zeros((L,), jnp.float32))

    def inter_body(it, acc):
        e = it * L + lanes
        b = e >> 1
        j = e & 1
        t1 = _gather(bt_v, b)
        ci = b * 4 + j * 2
        b2i = _gather(conns_v, ci)
        c2 = _gather(conns_v, ci + 1) & 1
        t2 = _gather(bt_v, b2i)
        a1 = _gather(paths0_v, t1 * 2 + j)
        a2 = _gather(paths0_v, t2 * 2 + c2)
        g1 = (_gather(offs_v, b) + a1) * 3
        g2 = (_gather(offs_v, b2i) + a2) * 3
        d2 = eps
        for c in range(3):
            dc = _gather(coords_v, g2 + c) - _gather(coords_v, g1 + c)
            d2 = d2 + dc * dc
        dd = _vsqrt(d2) - jnp.float32(1.5)
        return acc + jnp.float32(0.5) * dd * dd

    acc = lax.fori_loop(0, (B * 2) // L, inter_body, acc)

    total = jnp.sum(acc)
    res_v[...] = jnp.where(lanes == 0, jnp.broadcast_to(total, (L,)),
                           jnp.float32(0.0))
    pltpu.sync_copy(res_v, out_h.at[wid])


@jax.jit
def _run(coords2, bt, conns2, offs, subsT, uidf, widf, paths0, cnts, htab):
    mesh = plsc.VectorSubcoreMesh(core_axis_name="c", subcore_axis_name="s")
    f = pl.kernel(
        _body,
        out_type=jax.ShapeDtypeStruct((P, L), jnp.float32),
        mesh=mesh,
        compiler_params=pltpu.CompilerParams(needs_layout_passes=False,
                                             use_tc_tiling_on_sc=False),
        scratch_types=[
            pltpu.VMEM((N * 3,), jnp.float32),
            pltpu.VMEM((B,), jnp.int32),
            pltpu.VMEM((B * 4,), jnp.int32),
            pltpu.VMEM((B,), jnp.int32),
            pltpu.VMEM((4 * T * S,), jnp.int32),
            pltpu.VMEM((T * A,), jnp.int32),
            pltpu.VMEM((T * A,), jnp.int32),
            pltpu.VMEM((T * 2,), jnp.int32),
            pltpu.VMEM((T,), jnp.int32),
            pltpu.VMEM((T * S,), jnp.int32),
            pltpu.VMEM((T * S,), jnp.int32),
            pltpu.VMEM((T * S,), jnp.int32),
            pltpu.VMEM((T * S, 8), jnp.float32),
            pltpu.VMEM((T * S, 8), jnp.float32),
            pltpu.VMEM((8 * T * S,), jnp.float32),
            pltpu.VMEM((L,), jnp.float32),
            pltpu.SemaphoreType.DMA,
        ],
    )
    return f(coords2, bt, conns2, offs, subsT, uidf, widf, paths0, cnts, htab)


def kernel(coords, pose_stack_block_coord_offset, pose_stack_block_types,
           pose_stack_inter_block_connections, atom_paths_from_conn,
           atom_unique_ids, atom_wildcard_ids, hash_keys, hash_values,
           cart_subgraphs, cart_subgraph_offsets, max_subgraphs_per_block):
    coords2 = coords.reshape(P, N * 3)
    conns2 = pose_stack_inter_block_connections.reshape(P, B * 4)
    subsT = cart_subgraphs.transpose(2, 0, 1).reshape(4 * T * S)
    uidf = atom_unique_ids.reshape(T * A)
    widf = atom_wildcard_ids.reshape(T * A)
    paths0 = atom_paths_from_conn[:, :, 0].reshape(T * 2)
    htab = jnp.concatenate(
        [hash_values,
         lax.bitcast_convert_type(hash_keys, jnp.float32)[:, None],
         jnp.zeros((H, 1), jnp.float32)], axis=1)
    out = _run(coords2, pose_stack_block_types, conns2,
               pose_stack_block_coord_offset, subsT, uidf, widf, paths0,
               cart_subgraph_offsets, htab)
    return out[:, 0]


# speedup vs baseline: 3.5147x; 1.1175x over previous
import jax
import jax.numpy as jnp
from jax import lax
from jax.experimental import pallas as pl
from jax.experimental.pallas import tpu as pltpu
from jax.experimental.pallas import tpu_sc as plsc

P, B, A, T, S, H = 32, 256, 32, 32, 32, 16384
N = B * A
L = 16
PI = 3.14159265358979

_ACOS_C = (1.5707963050, -0.2145988016, 0.0889789874, -0.0501743046,
           0.0308918810, -0.0170881256, 0.0066700901, -0.0012624911)


def _iota():
    return lax.iota(jnp.int32, L)


def _splat_i(x):
    return jnp.broadcast_to(jnp.asarray(x, jnp.int32), (L,))


def _vrsqrt(x):
    i = lax.bitcast_convert_type(x, jnp.int32)
    y = lax.bitcast_convert_type(jnp.int32(0x5F3759DF) - (i >> 1), jnp.float32)
    half = jnp.float32(0.5) * x
    for _ in range(2):
        y = y * (jnp.float32(1.5) - half * y * y)
    return y


def _vsqrt(x):
    return x * _vrsqrt(x)


def _vacos(c):
    t = jnp.abs(c)
    p = jnp.full((L,), _ACOS_C[7], jnp.float32)
    for a in _ACOS_C[6::-1]:
        p = p * t + jnp.float32(a)
    pos = _vsqrt(jnp.float32(1.0) - t) * p
    return jnp.where(c >= 0, pos, jnp.float32(PI) - pos)


def _cossin_2piv(v):
    a = v * jnp.float32(4.0)
    q = a.astype(jnp.int32)
    z = (a - q.astype(jnp.float32)) * jnp.float32(PI / 2)
    z2 = z * z
    c0 = jnp.float32(1.0) + z2 * (jnp.float32(-0.5) + z2 * (
        jnp.float32(1.0 / 24) + z2 * (jnp.float32(-1.0 / 720)
                                      + z2 * jnp.float32(1.0 / 40320))))
    s0 = z * (jnp.float32(1.0) + z2 * (jnp.float32(-1.0 / 6) + z2 * (
        jnp.float32(1.0 / 120) + z2 * (jnp.float32(-1.0 / 5040)
                                       + z2 * jnp.float32(1.0 / 362880)))))
    q1, q2, q3 = q == 1, q == 2, q == 3
    cos = jnp.where(q1, -s0, jnp.where(q2, -c0, jnp.where(q3, s0, c0)))
    sin = jnp.where(q1, c0, jnp.where(q2, -s0, jnp.where(q3, -c0, s0)))
    return cos, sin


def _gather(ref, idx):
    return plsc.load_gather(ref, [idx])


def _body(cflat_h, bt_h, conns_h, subsT_h, uid_h, wid_h, paths0_h,
          cnt_h, hkeys_h, htab_h, out_h,
          cx_v, cy_v, cz_v, bt_v, conns_v, subsT_v, uid_v, wid_v, paths0_v,
          cnt_v, hk_v, ku_v, hu_v, hw_v, pu_v, pw_v, prmT_v, res_v, sem):
    wid = lax.axis_index("s") * 2 + lax.axis_index("c")

    pltpu.sync_copy(cflat_h.at[pl.ds(0 * P * N + wid * N, N)], cx_v)
    pltpu.sync_copy(cflat_h.at[pl.ds(1 * P * N + wid * N, N)], cy_v)
    pltpu.sync_copy(cflat_h.at[pl.ds(2 * P * N + wid * N, N)], cz_v)
    pltpu.sync_copy(bt_h.at[pl.ds(wid * B, B)], bt_v)
    pltpu.sync_copy(conns_h.at[pl.ds(wid * B * 4, B * 4)], conns_v)
    pltpu.sync_copy(subsT_h, subsT_v)
    pltpu.sync_copy(uid_h, uid_v)
    pltpu.sync_copy(wid_h, wid_v)
    pltpu.sync_copy(paths0_h, paths0_v)
    pltpu.sync_copy(cnt_h, cnt_v)
    pltpu.sync_copy(hkeys_h, hk_v)

    lanes = _iota()
    eps = jnp.float32(1e-6)

    def keys_body(i, carry):
        ts = i * L + lanes
        t32 = (ts >> 5) * 32
        ku = _splat_i(0)
        kw = _splat_i(0)
        for k, mult in enumerate((131, 31, 7, 1)):
            sub_k = _gather(subsT_v, k * 1024 + ts)
            ku = ku + _gather(uid_v, t32 + sub_k) * mult
            kw = kw + _gather(wid_v, t32 + sub_k) * mult
        plsc.store_scatter(ku_v, [ts], ku)
        plsc.store_scatter(hu_v, [ts], ku & (H - 1))
        plsc.store_scatter(hw_v, [ts], kw & (H - 1))
        return carry

    lax.fori_loop(0, (T * S) // L, keys_body, 0)

    copies = []
    for j in range(8):
        sl = pl.ds(j * 128, 128)
        copies.append(pltpu.async_copy(htab_h.at[hu_v.at[sl]], pu_v.at[sl], sem))
        copies.append(pltpu.async_copy(htab_h.at[hw_v.at[sl]], pw_v.at[sl], sem))
    for c in copies:
        c.wait()

    def prm_body(i, carry):
        ts = i * L + lanes
        t = ts >> 5
        s = ts & 31
        ku = _gather(ku_v, ts)
        hu = _gather(hu_v, ts)
        match = _gather(hk_v, hu) == ku
        prm = []
        for c in range(6):
            pu_c = plsc.load_gather(pu_v, [ts, _splat_i(c)])
            pw_c = plsc.load_gather(pw_v, [ts, _splat_i(c)])
            prm.append(jnp.where(match, pu_c, pw_c))
        cp0, sp0 = _cossin_2piv(prm[5])
        cntv = _gather(cnt_v, t)
        maskf = jnp.where(s < cntv, jnp.float32(1.0), jnp.float32(0.0))
        rows = (prm[0], prm[1] * jnp.float32(2.0), prm[2],
                prm[3] * jnp.float32(PI), prm[4], cp0, sp0, maskf)
        for c, val in enumerate(rows):
            plsc.store_scatter(prmT_v, [c * 1024 + ts], val)
        return carry

    lax.fori_loop(0, (T * S) // L, prm_body, 0)

    def intra_body(q, acc):
        b = q >> 1
        s0 = (q & 1) * L
        bvec = jnp.broadcast_to(b, (L,))
        tvec = _gather(bt_v, bvec)
        ovec = bvec * 32
        ts = tvec * 32 + s0 + lanes
        xs = []
        for k in range(4):
            sub_k = _gather(subsT_v, k * 1024 + ts)
            gk = ovec + sub_k
            xs.append((_gather(cx_v, gk), _gather(cy_v, gk),
                       _gather(cz_v, gk)))
        x0, x1, x2, x3 = xs
        prm = tuple(_gather(prmT_v, c * 1024 + ts) for c in range(8))
        k_len, l0, k_ang, t0, k_tor, cp0, sp0, maskf = prm

        dx = tuple(x1[c] - x0[c] for c in range(3))
        d01 = _vsqrt(dx[0] * dx[0] + dx[1] * dx[1] + dx[2] * dx[2] + eps)
        uv = tuple(x0[c] - x1[c] for c in range(3))
        vv = tuple(x2[c] - x1[c] for c in range(3))
        s_uv = uv[0] * uv[0] + uv[1] * uv[1] + uv[2] * uv[2] + eps
        s_vv = vv[0] * vv[0] + vv[1] * vv[1] + vv[2] * vv[2] + eps
        dotuv = uv[0] * vv[0] + uv[1] * vv[1] + uv[2] * vv[2]
        cosang = jnp.clip(dotuv * _vrsqrt(s_uv * s_vv),
                          jnp.float32(-1.0 + 1e-6), jnp.float32(1.0 - 1e-6))
        theta = _vacos(cosang)
        b1 = dx
        b2 = vv
        b3 = tuple(x3[c] - x2[c] for c in range(3))

        def cross(u, v):
            return (u[1] * v[2] - u[2] * v[1],
                    u[2] * v[0] - u[0] * v[2],
                    u[0] * v[1] - u[1] * v[0])

        n1 = cross(b1, b2)
        n2 = cross(b2, b3)
        s_b2 = b2[0] * b2[0] + b2[1] * b2[1] + b2[2] * b2[2]
        inv_b2 = jnp.float32(1.0) / (_vsqrt(s_b2) + eps)
        m1 = cross(n1, tuple(b2[c] * inv_b2 for c in range(3)))
        y = m1[0] * n2[0] + m1[1] * n2[1] + m1[2] * n2[2]
        x = n1[0] * n2[0] + n1[1] * n2[1] + n1[2] * n2[2] + eps
        den = x * x + y * y + jnp.float32(1e-30)
        cos2phi = (x * x - y * y) / den
        sin2phi = jnp.float32(2.0) * x * y / den

        dl = d01 - l0
        da = theta - t0
        E = (k_len * dl * dl + k_ang * da * da
             + k_tor * (jnp.float32(1.0) + cos2phi * cp0 + sin2phi * sp0))
        return acc + E * maskf

    acc = lax.fori_loop(0, (B * S) // L, intra_body,
                        jnp.zeros((L,), jnp.float32))

    def inter_body(it, acc):
        e = it * L + lanes
        b = e >> 1
        j = e & 1
        t1 = _gather(bt_v, b)
        ci = b * 4 + j * 2
        b2i = _gather(conns_v, ci)
        c2 = _gather(conns_v, ci + 1) & 1
        t2 = _gather(bt_v, b2i)
        a1 = _gather(paths0_v, t1 * 2 + j)
        a2 = _gather(paths0_v, t2 * 2 + c2)
        g1 = b * 32 + a1
        g2 = b2i * 32 + a2
        d2 = eps
        for cv in (cx_v, cy_v, cz_v):
            dc = _gather(cv, g2) - _gather(cv, g1)
            d2 = d2 + dc * dc
        dd = _vsqrt(d2) - jnp.float32(1.5)
        return acc + jnp.float32(0.5) * dd * dd

    acc = lax.fori_loop(0, (B * 2) // L, inter_body, acc)

    total = jnp.sum(acc)
    res_v[...] = jnp.where(lanes == 0, jnp.broadcast_to(total, (L,)),
                           jnp.float32(0.0))
    pltpu.sync_copy(res_v, out_h.at[wid])


@jax.jit
def _run(cflat, bt1, conns1, subsT, uidf, widf, paths0, cnts, hkeys, htab):
    mesh = plsc.VectorSubcoreMesh(core_axis_name="c", subcore_axis_name="s")
    f = pl.kernel(
        _body,
        out_type=jax.ShapeDtypeStruct((P, L), jnp.float32),
        mesh=mesh,
        compiler_params=pltpu.CompilerParams(needs_layout_passes=False,
                                             use_tc_tiling_on_sc=False),
        scratch_types=[
            pltpu.VMEM((N,), jnp.float32),
            pltpu.VMEM((N,), jnp.float32),
            pltpu.VMEM((N,), jnp.float32),
            pltpu.VMEM((B,), jnp.int32),
            pltpu.VMEM((B * 4,), jnp.int32),
            pltpu.VMEM((4 * T * S,), jnp.int32),
            pltpu.VMEM((T * A,), jnp.int32),
            pltpu.VMEM((T * A,), jnp.int32),
            pltpu.VMEM((T * 2,), jnp.int32),
            pltpu.VMEM((T,), jnp.int32),
            pltpu.VMEM((H,), jnp.int32),
            pltpu.VMEM((T * S,), jnp.int32),
            pltpu.VMEM((T * S,), jnp.int32),
            pltpu.VMEM((T * S,), jnp.int32),
            pltpu.VMEM((T * S, 8), jnp.float32),
            pltpu.VMEM((T * S, 8), jnp.float32),
            pltpu.VMEM((8 * T * S,), jnp.float32),
            pltpu.VMEM((L,), jnp.float32),
            pltpu.SemaphoreType.DMA,
        ],
    )
    return f(cflat, bt1, conns1, subsT, uidf, widf, paths0, cnts, hkeys, htab)


def kernel(coords, pose_stack_block_coord_offset, pose_stack_block_types,
           pose_stack_inter_block_connections, atom_paths_from_conn,
           atom_unique_ids, atom_wildcard_ids, hash_keys, hash_values,
           cart_subgraphs, cart_subgraph_offsets, max_subgraphs_per_block):
    cflat = coords.transpose(2, 0, 1).reshape(3 * P * N)
    bt1 = pose_stack_block_types.reshape(P * B)
    conns1 = pose_stack_inter_block_connections.reshape(P * B * 4)
    subsT = cart_subgraphs.transpose(2, 0, 1).reshape(4 * T * S)
    uidf = atom_unique_ids.reshape(T * A)
    widf = atom_wildcard_ids.reshape(T * A)
    paths0 = atom_paths_from_conn[:, :, 0].reshape(T * 2)
    htab = jnp.concatenate(
        [hash_values,
         lax.bitcast_convert_type(hash_keys, jnp.float32)[:, None],
         jnp.zeros((H, 1), jnp.float32)], axis=1)
    out = _run(cflat, bt1, conns1, subsT, uidf, widf, paths0,
               cart_subgraph_offsets, hash_keys, htab)
    return out[:, 0]
